# baseline (speedup 1.0000x reference)
import functools

import jax
import jax.numpy as jnp
from jax import lax
from jax.experimental import pallas as pl
from jax.experimental.pallas import tpu as pltpu
from jax.experimental.pallas import tpu_sc as plsc

_NUM_CLASSES = 1000
_HIDDEN = 1152
_BATCH = 16384
_ROWS = _NUM_CLASSES + 1

_NC = 2
_NS = 16
_GRP = 16
_NG0 = 40
_NG1 = 24
_BPW0 = _NG0 * _GRP
_BPW1 = _NG1 * _GRP
_SPLIT = _NS * _BPW0

_STG = 64
_STG_LAST = _ROWS - 15 * _STG

_mesh = plsc.VectorSubcoreMesh(core_axis_name="c", subcore_axis_name="s")


@functools.partial(
    pl.kernel,
    mesh=_mesh,
    out_type=jax.ShapeDtypeStruct((_BATCH, _HIDDEN), jnp.float32),
    scratch_types=[
        pltpu.VMEM_SHARED((_ROWS * _HIDDEN,), jnp.float32),
        pltpu.VMEM((2, _GRP, _HIDDEN), jnp.float32),
        pltpu.VMEM((_BPW0,), jnp.int32),
        pltpu.VMEM((_BPW0,), jnp.int32),
        pltpu.SemaphoreType.DMA,
        pltpu.SemaphoreType.DMA,
        pltpu.SemaphoreType.DMA,
    ],
)
def _embed(labels_hbm, force_hbm, table_hbm, out_hbm,
           table_sp, rowbuf, idx_v, frc_v, rsem, osem, ssem):
    cid = lax.axis_index("c")
    sid = lax.axis_index("s")

    @pl.when(sid < 15)
    def _():
        off = pl.multiple_of(sid * _STG * _HIDDEN, _STG * _HIDDEN)
        pltpu.async_copy(table_hbm.at[pl.ds(off, _STG * _HIDDEN)],
                         table_sp.at[pl.ds(off, _STG * _HIDDEN)], ssem)

    @pl.when(sid == 15)
    def _():
        pltpu.async_copy(
            table_hbm.at[pl.ds(15 * _STG * _HIDDEN, _STG_LAST * _HIDDEN)],
            table_sp.at[pl.ds(15 * _STG * _HIDDEN, _STG_LAST * _HIDDEN)], ssem)

    def prep_indices(base, bpw):
        pltpu.sync_copy(labels_hbm.at[pl.ds(base, bpw)], idx_v.at[pl.ds(0, bpw)])
        pltpu.sync_copy(force_hbm.at[pl.ds(base, bpw)], frc_v.at[pl.ds(0, bpw)])
        for i in range(bpw // 16):
            sl = pl.ds(i * 16, 16)
            idx_v[sl] = jnp.where(frc_v[sl] == 1, _NUM_CLASSES, idx_v[sl])

    @pl.when(cid == 0)
    def _():
        prep_indices(pl.multiple_of(sid * _BPW0, _GRP), _BPW0)

    @pl.when(cid == 1)
    def _():
        prep_indices(pl.multiple_of(_SPLIT + sid * _BPW1, _GRP), _BPW1)

    @pl.when(sid < 15)
    def _():
        off = pl.multiple_of(sid * _STG * _HIDDEN, _STG * _HIDDEN)
        pltpu.make_async_copy(table_hbm.at[pl.ds(off, _STG * _HIDDEN)],
                              table_sp.at[pl.ds(off, _STG * _HIDDEN)],
                              ssem).wait()

    @pl.when(sid == 15)
    def _():
        pltpu.make_async_copy(
            table_hbm.at[pl.ds(15 * _STG * _HIDDEN, _STG_LAST * _HIDDEN)],
            table_sp.at[pl.ds(15 * _STG * _HIDDEN, _STG_LAST * _HIDDEN)],
            ssem).wait()

    plsc.subcore_barrier()

    def gather_group(g, b):
        idx16 = idx_v[pl.ds(pl.multiple_of(g * _GRP, _GRP), _GRP)]
        for r in range(_GRP):
            off = pl.multiple_of(idx16[r] * _HIDDEN, _HIDDEN)
            pltpu.async_copy(table_sp.at[pl.ds(off, _HIDDEN)],
                             rowbuf.at[b, r], rsem)
        pltpu.make_async_copy(
            out_hbm.at[pl.ds(0, _GRP)], rowbuf.at[b], rsem).wait()

    def run(base, ngrp):
        @pl.loop(0, ngrp, step=2)
        def _(g0):
            for b in range(2):
                g = g0 + b

                @pl.when(g >= 2)
                def _():
                    pltpu.make_async_copy(
                        out_hbm.at[pl.ds(0, _GRP)], rowbuf.at[b], osem).wait()

                gather_group(g, b)
                pltpu.async_copy(
                    rowbuf.at[b],
                    out_hbm.at[pl.ds(base + g * _GRP, _GRP)], osem)

        for b in range(2):
            pltpu.make_async_copy(
                out_hbm.at[pl.ds(0, _GRP)], rowbuf.at[b], osem).wait()

    @pl.when(cid == 0)
    def _():
        run(pl.multiple_of(sid * _BPW0, _GRP), _NG0)

    @pl.when(cid == 1)
    def _():
        run(pl.multiple_of(_SPLIT + sid * _BPW1, _GRP), _NG1)


def kernel(labels, train, force_drop_ids, embedding_table):
    del train
    return _embed(labels.astype(jnp.int32),
                  force_drop_ids.astype(jnp.int32),
                  embedding_table.reshape(-1))


# speedup vs baseline: 1.0109x; 1.0109x over previous
import functools

import jax
import jax.numpy as jnp
from jax import lax
from jax.experimental import pallas as pl
from jax.experimental.pallas import tpu as pltpu
from jax.experimental.pallas import tpu_sc as plsc

_NUM_CLASSES = 1000
_HIDDEN = 1152
_BATCH = 16384
_ROWS = _NUM_CLASSES + 1

_NC = 2
_NS = 16
_GRP = 16
_NG0 = 24
_NG1 = 40
_BPW0 = _NG0 * _GRP
_BPW1 = _NG1 * _GRP
_SPLIT = _NS * _BPW0

_STG = 64
_STG_LAST = _ROWS - 15 * _STG

_mesh = plsc.VectorSubcoreMesh(core_axis_name="c", subcore_axis_name="s")


@functools.partial(
    pl.kernel,
    mesh=_mesh,
    out_type=jax.ShapeDtypeStruct((_BATCH, _HIDDEN), jnp.float32),
    scratch_types=[
        pltpu.VMEM_SHARED((_ROWS * _HIDDEN,), jnp.float32),
        pltpu.VMEM((2, _GRP, _HIDDEN), jnp.float32),
        pltpu.VMEM((max(_BPW0, _BPW1),), jnp.int32),
        pltpu.VMEM((max(_BPW0, _BPW1),), jnp.int32),
        pltpu.SemaphoreType.DMA,
        pltpu.SemaphoreType.DMA,
        pltpu.SemaphoreType.DMA,
    ],
)
def _embed(labels_hbm, force_hbm, table_hbm, out_hbm,
           table_sp, rowbuf, idx_v, frc_v, rsem, osem, ssem):
    cid = lax.axis_index("c")
    sid = lax.axis_index("s")

    @pl.when(sid < 15)
    def _():
        off = pl.multiple_of(sid * _STG * _HIDDEN, _STG * _HIDDEN)
        pltpu.async_copy(table_hbm.at[pl.ds(off, _STG * _HIDDEN)],
                         table_sp.at[pl.ds(off, _STG * _HIDDEN)], ssem)

    @pl.when(sid == 15)
    def _():
        pltpu.async_copy(
            table_hbm.at[pl.ds(15 * _STG * _HIDDEN, _STG_LAST * _HIDDEN)],
            table_sp.at[pl.ds(15 * _STG * _HIDDEN, _STG_LAST * _HIDDEN)], ssem)

    def prep_indices(base, bpw):
        pltpu.sync_copy(labels_hbm.at[pl.ds(base, bpw)], idx_v.at[pl.ds(0, bpw)])
        pltpu.sync_copy(force_hbm.at[pl.ds(base, bpw)], frc_v.at[pl.ds(0, bpw)])
        for i in range(bpw // 16):
            sl = pl.ds(i * 16, 16)
            idx_v[sl] = jnp.where(frc_v[sl] == 1, _NUM_CLASSES, idx_v[sl])

    @pl.when(cid == 0)
    def _():
        prep_indices(pl.multiple_of(sid * _BPW0, _GRP), _BPW0)

    @pl.when(cid == 1)
    def _():
        prep_indices(pl.multiple_of(_SPLIT + sid * _BPW1, _GRP), _BPW1)

    @pl.when(sid < 15)
    def _():
        off = pl.multiple_of(sid * _STG * _HIDDEN, _STG * _HIDDEN)
        pltpu.make_async_copy(table_hbm.at[pl.ds(off, _STG * _HIDDEN)],
                              table_sp.at[pl.ds(off, _STG * _HIDDEN)],
                              ssem).wait()

    @pl.when(sid == 15)
    def _():
        pltpu.make_async_copy(
            table_hbm.at[pl.ds(15 * _STG * _HIDDEN, _STG_LAST * _HIDDEN)],
            table_sp.at[pl.ds(15 * _STG * _HIDDEN, _STG_LAST * _HIDDEN)],
            ssem).wait()

    plsc.subcore_barrier()

    def gather_group(g, b):
        idx16 = idx_v[pl.ds(pl.multiple_of(g * _GRP, _GRP), _GRP)]
        for r in range(_GRP):
            off = pl.multiple_of(idx16[r] * _HIDDEN, _HIDDEN)
            pltpu.async_copy(table_sp.at[pl.ds(off, _HIDDEN)],
                             rowbuf.at[b, r], rsem)
        pltpu.make_async_copy(
            out_hbm.at[pl.ds(0, _GRP)], rowbuf.at[b], rsem).wait()

    def run(base, ngrp):
        @pl.loop(0, ngrp, step=2)
        def _(g0):
            for b in range(2):
                g = g0 + b

                @pl.when(g >= 2)
                def _():
                    pltpu.make_async_copy(
                        out_hbm.at[pl.ds(0, _GRP)], rowbuf.at[b], osem).wait()

                gather_group(g, b)
                pltpu.async_copy(
                    rowbuf.at[b],
                    out_hbm.at[pl.ds(base + g * _GRP, _GRP)], osem)

        for b in range(2):
            pltpu.make_async_copy(
                out_hbm.at[pl.ds(0, _GRP)], rowbuf.at[b], osem).wait()

    @pl.when(cid == 0)
    def _():
        run(pl.multiple_of(sid * _BPW0, _GRP), _NG0)

    @pl.when(cid == 1)
    def _():
        run(pl.multiple_of(_SPLIT + sid * _BPW1, _GRP), _NG1)


def kernel(labels, train, force_drop_ids, embedding_table):
    del train
    return _embed(labels.astype(jnp.int32),
                  force_drop_ids.astype(jnp.int32),
                  embedding_table.reshape(-1))


# speedup vs baseline: 1.1780x; 1.1652x over previous
import functools

import jax
import jax.numpy as jnp
from jax import lax
from jax.experimental import pallas as pl
from jax.experimental.pallas import tpu as pltpu
from jax.experimental.pallas import tpu_sc as plsc

_NUM_CLASSES = 1000
_HIDDEN = 1152
_BATCH = 16384
_ROWS = _NUM_CLASSES + 1

_NC = 2
_NS = 16
_NW = _NC * _NS
_BPW = _BATCH // _NW
_GRP = 16
_NGRP = _BPW // _GRP

_STG = 64
_STG_LAST = _ROWS - 15 * _STG

_mesh = plsc.VectorSubcoreMesh(core_axis_name="c", subcore_axis_name="s")


@functools.partial(
    pl.kernel,
    mesh=_mesh,
    out_type=jax.ShapeDtypeStruct((_BATCH, _HIDDEN), jnp.float32),
    scratch_types=[
        pltpu.VMEM_SHARED((_ROWS * _HIDDEN,), jnp.float32),
        pltpu.VMEM((2, _GRP, _HIDDEN), jnp.float32),
        pltpu.VMEM((_BPW,), jnp.int32),
        pltpu.VMEM((_BPW,), jnp.int32),
        pltpu.SemaphoreType.DMA,
        pltpu.SemaphoreType.DMA,
        pltpu.SemaphoreType.DMA,
    ],
)
def _embed(labels_hbm, force_hbm, table_hbm, out_hbm,
           table_sp, rowbuf, idx_v, frc_v, rsem, osem, ssem):
    cid = lax.axis_index("c")
    sid = lax.axis_index("s")
    wid = sid * _NC + cid
    base = pl.multiple_of(wid * _BPW, _BPW)

    @pl.when(sid < 15)
    def _():
        off = pl.multiple_of(sid * _STG * _HIDDEN, _STG * _HIDDEN)
        pltpu.async_copy(table_hbm.at[pl.ds(off, _STG * _HIDDEN)],
                         table_sp.at[pl.ds(off, _STG * _HIDDEN)], ssem)

    @pl.when(sid == 15)
    def _():
        pltpu.async_copy(
            table_hbm.at[pl.ds(15 * _STG * _HIDDEN, _STG_LAST * _HIDDEN)],
            table_sp.at[pl.ds(15 * _STG * _HIDDEN, _STG_LAST * _HIDDEN)], ssem)

    pltpu.sync_copy(labels_hbm.at[pl.ds(base, _BPW)], idx_v)
    pltpu.sync_copy(force_hbm.at[pl.ds(base, _BPW)], frc_v)

    for i in range(_BPW // 16):
        sl = pl.ds(i * 16, 16)
        idx_v[sl] = jnp.where(frc_v[sl] == 1, _NUM_CLASSES, idx_v[sl])

    @pl.when(sid < 15)
    def _():
        off = pl.multiple_of(sid * _STG * _HIDDEN, _STG * _HIDDEN)
        pltpu.make_async_copy(table_hbm.at[pl.ds(off, _STG * _HIDDEN)],
                              table_sp.at[pl.ds(off, _STG * _HIDDEN)],
                              ssem).wait()

    @pl.when(sid == 15)
    def _():
        pltpu.make_async_copy(
            table_hbm.at[pl.ds(15 * _STG * _HIDDEN, _STG_LAST * _HIDDEN)],
            table_sp.at[pl.ds(15 * _STG * _HIDDEN, _STG_LAST * _HIDDEN)],
            ssem).wait()

    plsc.subcore_barrier()

    def gather_group(g, b):
        idx16 = idx_v[pl.ds(pl.multiple_of(g * _GRP, _GRP), _GRP)]
        for r in range(_GRP):
            off = pl.multiple_of(idx16[r] * _HIDDEN, _HIDDEN)
            pltpu.async_copy(table_sp.at[pl.ds(off, _HIDDEN)],
                             rowbuf.at[b, r], rsem)
        pltpu.make_async_copy(
            out_hbm.at[pl.ds(0, _GRP)], rowbuf.at[b], rsem).wait()

    @pl.loop(0, _NGRP, step=2)
    def _(g0):
        for b in range(2):
            g = g0 + b

            @pl.when(g >= 2)
            def _():
                pltpu.make_async_copy(
                    out_hbm.at[pl.ds(0, _GRP)], rowbuf.at[b], osem).wait()

            gather_group(g, b)
            pltpu.async_copy(
                rowbuf.at[b], out_hbm.at[pl.ds(base + g * _GRP, _GRP)], osem)

    for b in range(2):
        pltpu.make_async_copy(
            out_hbm.at[pl.ds(0, _GRP)], rowbuf.at[b], osem).wait()


def kernel(labels, train, force_drop_ids, embedding_table):
    del train
    return _embed(labels.astype(jnp.int32),
                  force_drop_ids.astype(jnp.int32),
                  embedding_table.reshape(-1))


# speedup vs baseline: 1.1829x; 1.0042x over previous
import functools

import jax
import jax.numpy as jnp
from jax import lax
from jax.experimental import pallas as pl
from jax.experimental.pallas import tpu as pltpu
from jax.experimental.pallas import tpu_sc as plsc

_NUM_CLASSES = 1000
_HIDDEN = 1152
_BATCH = 16384
_ROWS = _NUM_CLASSES + 1

_NC = 2
_NS = 16
_NW = _NC * _NS
_BPW = _BATCH // _NW
_GRP = 8
_NGRP = _BPW // _GRP

_STG = 64
_STG_LAST = _ROWS - 15 * _STG

_mesh = plsc.VectorSubcoreMesh(core_axis_name="c", subcore_axis_name="s")


@functools.partial(
    pl.kernel,
    mesh=_mesh,
    out_type=jax.ShapeDtypeStruct((_BATCH, _HIDDEN), jnp.float32),
    scratch_types=[
        pltpu.VMEM_SHARED((_ROWS * _HIDDEN,), jnp.float32),
        pltpu.VMEM((2, _GRP, _HIDDEN), jnp.float32),
        pltpu.VMEM((_BPW,), jnp.int32),
        pltpu.VMEM((_BPW,), jnp.int32),
        pltpu.SemaphoreType.DMA,
        pltpu.SemaphoreType.DMA,
        pltpu.SemaphoreType.DMA,
    ],
)
def _embed(labels_hbm, force_hbm, table_hbm, out_hbm,
           table_sp, rowbuf, idx_v, frc_v, rsem, osem, ssem):
    cid = lax.axis_index("c")
    sid = lax.axis_index("s")
    wid = sid * _NC + cid
    base = pl.multiple_of(wid * _BPW, _BPW)

    @pl.when(sid < 15)
    def _():
        off = pl.multiple_of(sid * _STG * _HIDDEN, _STG * _HIDDEN)
        pltpu.async_copy(table_hbm.at[pl.ds(off, _STG * _HIDDEN)],
                         table_sp.at[pl.ds(off, _STG * _HIDDEN)], ssem)

    @pl.when(sid == 15)
    def _():
        pltpu.async_copy(
            table_hbm.at[pl.ds(15 * _STG * _HIDDEN, _STG_LAST * _HIDDEN)],
            table_sp.at[pl.ds(15 * _STG * _HIDDEN, _STG_LAST * _HIDDEN)], ssem)

    pltpu.sync_copy(labels_hbm.at[pl.ds(base, _BPW)], idx_v)
    pltpu.sync_copy(force_hbm.at[pl.ds(base, _BPW)], frc_v)

    for i in range(_BPW // 16):
        sl = pl.ds(i * 16, 16)
        idx_v[sl] = jnp.where(frc_v[sl] == 1, _NUM_CLASSES, idx_v[sl])

    @pl.when(sid < 15)
    def _():
        off = pl.multiple_of(sid * _STG * _HIDDEN, _STG * _HIDDEN)
        pltpu.make_async_copy(table_hbm.at[pl.ds(off, _STG * _HIDDEN)],
                              table_sp.at[pl.ds(off, _STG * _HIDDEN)],
                              ssem).wait()

    @pl.when(sid == 15)
    def _():
        pltpu.make_async_copy(
            table_hbm.at[pl.ds(15 * _STG * _HIDDEN, _STG_LAST * _HIDDEN)],
            table_sp.at[pl.ds(15 * _STG * _HIDDEN, _STG_LAST * _HIDDEN)],
            ssem).wait()

    plsc.subcore_barrier()

    def gather_group(g, b):
        idx16 = idx_v[pl.ds(pl.multiple_of(g * _GRP, _GRP), _GRP)]
        for r in range(_GRP):
            off = pl.multiple_of(idx16[r] * _HIDDEN, _HIDDEN)
            pltpu.async_copy(table_sp.at[pl.ds(off, _HIDDEN)],
                             rowbuf.at[b, r], rsem)
        pltpu.make_async_copy(
            out_hbm.at[pl.ds(0, _GRP)], rowbuf.at[b], rsem).wait()

    @pl.loop(0, _NGRP, step=2)
    def _(g0):
        for b in range(2):
            g = g0 + b

            @pl.when(g >= 2)
            def _():
                pltpu.make_async_copy(
                    out_hbm.at[pl.ds(0, _GRP)], rowbuf.at[b], osem).wait()

            gather_group(g, b)
            pltpu.async_copy(
                rowbuf.at[b], out_hbm.at[pl.ds(base + g * _GRP, _GRP)], osem)

    for b in range(2):
        pltpu.make_async_copy(
            out_hbm.at[pl.ds(0, _GRP)], rowbuf.at[b], osem).wait()


def kernel(labels, train, force_drop_ids, embedding_table):
    del train
    return _embed(labels.astype(jnp.int32),
                  force_drop_ids.astype(jnp.int32),
                  embedding_table.reshape(-1))
